# baseline (speedup 1.0000x reference)
import functools

import jax
import jax.numpy as jnp
from jax import lax
from jax.experimental import pallas as pl
from jax.experimental.pallas import tpu as pltpu
from jax.experimental.pallas import tpu_sc as plsc

E = 8
K = 2
D = 2048
F = 512
N = 4096
A = N * K
BLK = 256
NP = A + E * BLK
NB = NP // BLK
RB = 512

NW = 32
DISPATCH_ROWS = NP // NW
DISPATCH_CH = 16
TOK_PER_W = N // NW
COMBINE_CH = 16


def _router_body(x_ref, wr_ref, idx_ref, w_ref):
    x = x_ref[...]
    s = jnp.dot(x, wr_ref[...], preferred_element_type=jnp.float32)
    lane = lax.broadcasted_iota(jnp.int32, (RB, 128), 1)
    neg = jnp.float32(-jnp.inf)
    s = jnp.where(lane < E, s, neg)
    m0 = jnp.max(s, axis=1, keepdims=True)
    i0 = jnp.min(jnp.where(s == m0, lane, 128), axis=1, keepdims=True)
    s2 = jnp.where(lane == i0, neg, s)
    m1 = jnp.max(s2, axis=1, keepdims=True)
    i1 = jnp.min(jnp.where(s2 == m1, lane, 128), axis=1, keepdims=True)
    e1 = jnp.exp(m1 - m0)
    denom = 1.0 + e1
    w0 = 1.0 / denom
    w1 = e1 / denom
    idx_ref[...] = jnp.concatenate([i0, i1], axis=1)
    w_ref[...] = jnp.concatenate([w0, w1], axis=1)


def _run_router(x2, wr_pad):
    return pl.pallas_call(
        _router_body,
        grid=(N // RB,),
        in_specs=[
            pl.BlockSpec((RB, D), lambda i: (i, 0)),
            pl.BlockSpec((D, 128), lambda i: (0, 0)),
        ],
        out_specs=[
            pl.BlockSpec((RB, K), lambda i: (i, 0)),
            pl.BlockSpec((RB, K), lambda i: (i, 0)),
        ],
        out_shape=[
            jax.ShapeDtypeStruct((N, K), jnp.int32),
            jax.ShapeDtypeStruct((N, K), jnp.float32),
        ],
    )(x2, wr_pad)


NSTR = 4
NSTR_D = 2


def _dispatch_body(x_hbm, gsrc_hbm, out_hbm, ia, ib, ra, rb,
                   ga0, ga1, gb0, gb1, sa, sb):
    wid = lax.axis_index("s") * 2 + lax.axis_index("c")
    base = wid * DISPATCH_ROWS
    nch = DISPATCH_ROWS // DISPATCH_CH
    sub = DISPATCH_CH // NSTR_D
    idx = [ia, ib]
    rows = [ra, rb]
    gsem = [[ga0, ga1], [gb0, gb1]]
    ssem = [sa, sb]
    gcp = [None, None]
    scp = [None, None]

    def start_gather(b):
        cps = []
        for k in range(NSTR_D):
            cps.append(pltpu.async_copy(
                x_hbm.at[idx[b].at[pl.ds(k * sub, sub)]],
                rows[b].at[pl.ds(k * sub, sub)],
                gsem[b][k]))
        return cps

    pltpu.sync_copy(gsrc_hbm.at[pl.ds(base, DISPATCH_CH)], ia)
    gcp[0] = start_gather(0)
    for c in range(nch):
        b = c & 1
        nb = 1 - b
        if c + 1 < nch:
            if scp[nb] is not None:
                scp[nb].wait()
            noff = base + (c + 1) * DISPATCH_CH
            pltpu.sync_copy(gsrc_hbm.at[pl.ds(noff, DISPATCH_CH)], idx[nb])
            gcp[nb] = start_gather(nb)
        for cp in gcp[b]:
            cp.wait()
        off = base + c * DISPATCH_CH
        scp[b] = pltpu.async_copy(rows[b], out_hbm.at[pl.ds(off, DISPATCH_CH)],
                                  ssem[b])
    scp[0].wait()
    scp[1].wait()


def _run_dispatch(x2, gsrc):
    mesh = plsc.VectorSubcoreMesh(core_axis_name="c", subcore_axis_name="s")
    return pl.kernel(
        _dispatch_body,
        out_type=jax.ShapeDtypeStruct((NP, D), jnp.float32),
        mesh=mesh,
        scratch_types=[
            pltpu.VMEM((DISPATCH_CH,), jnp.int32),
            pltpu.VMEM((DISPATCH_CH,), jnp.int32),
            pltpu.VMEM((DISPATCH_CH, D), jnp.float32),
            pltpu.VMEM((DISPATCH_CH, D), jnp.float32),
        ] + [pltpu.SemaphoreType.DMA] * (2 * NSTR_D + 2),
    )(x2, gsrc)


def _gemm_body(be_ref, xg_ref, w1_ref, w3_ref, w2_ref, ws_ref, out_ref):
    del be_ref
    xb = xg_ref[...].astype(jnp.bfloat16)
    x1 = jnp.dot(xb, w1_ref[0], preferred_element_type=jnp.float32)
    x3 = jnp.dot(xb, w3_ref[0], preferred_element_type=jnp.float32)
    x1 = x1.astype(jnp.bfloat16).astype(jnp.float32)
    x3 = x3.astype(jnp.bfloat16).astype(jnp.float32)
    sg = jax.nn.sigmoid(x1).astype(jnp.bfloat16).astype(jnp.float32)
    sil = (x1 * sg).astype(jnp.bfloat16).astype(jnp.float32)
    h = (sil * x3).astype(jnp.bfloat16)
    out = jnp.dot(h, w2_ref[0], preferred_element_type=jnp.float32)
    out = out.astype(jnp.bfloat16).astype(jnp.float32)
    out_ref[...] = out * ws_ref[...]


def _run_gemm(be, xg2, w1b, w3b, w2b, ws2):
    grid_spec = pltpu.PrefetchScalarGridSpec(
        num_scalar_prefetch=1,
        grid=(NB,),
        in_specs=[
            pl.BlockSpec((BLK, D), lambda i, be: (i, 0)),
            pl.BlockSpec((1, D, F), lambda i, be: (be[i], 0, 0)),
            pl.BlockSpec((1, D, F), lambda i, be: (be[i], 0, 0)),
            pl.BlockSpec((1, F, D), lambda i, be: (be[i], 0, 0)),
            pl.BlockSpec((BLK, 1), lambda i, be: (i, 0)),
        ],
        out_specs=pl.BlockSpec((BLK, D), lambda i, be: (i, 0)),
    )
    return pl.pallas_call(
        _gemm_body,
        grid_spec=grid_spec,
        out_shape=jax.ShapeDtypeStruct((NP, D), jnp.float32),
        compiler_params=pltpu.CompilerParams(
            dimension_semantics=("arbitrary",),
        ),
    )(be, xg2, w1b, w3b, w2b, ws2)


def _combine_body(eo_hbm, dest_hbm, y_hbm, idx_v, rows_v, out_v,
                  g0, g1, g2, g3, ssem):
    wid = lax.axis_index("s") * 2 + lax.axis_index("c")
    base = wid * TOK_PER_W
    nch = TOK_PER_W // COMBINE_CH
    nrows = 2 * COMBINE_CH
    sub = nrows // NSTR
    gsem = [g0, g1, g2, g3]
    unroll = 8
    iters_per_row = (D // 16) // unroll
    scp = None

    for c in range(nch):
        off = base + c * COMBINE_CH
        pltpu.sync_copy(dest_hbm.at[pl.ds(2 * off, nrows)], idx_v)
        cps = [pltpu.async_copy(
            eo_hbm.at[idx_v.at[pl.ds(k * sub, sub)]],
            rows_v.at[pl.ds(k * sub, sub)],
            gsem[k]) for k in range(NSTR)]
        for cp in cps:
            cp.wait()
        if scp is not None:
            scp.wait()

        def body(i, carry):
            r = i // iters_per_row
            cb = (i % iters_per_row) * unroll
            for j in range(unroll):
                col = (cb + j) * 16
                out_v[r, pl.ds(col, 16)] = (
                    rows_v[2 * r, pl.ds(col, 16)]
                    + rows_v[2 * r + 1, pl.ds(col, 16)]
                )
            return carry

        lax.fori_loop(0, COMBINE_CH * iters_per_row, body, 0)
        scp = pltpu.async_copy(out_v, y_hbm.at[pl.ds(off, COMBINE_CH)], ssem)
    scp.wait()


def _run_combine(eo, dest):
    mesh = plsc.VectorSubcoreMesh(core_axis_name="c", subcore_axis_name="s")
    return pl.kernel(
        _combine_body,
        out_type=jax.ShapeDtypeStruct((N, D), jnp.float32),
        mesh=mesh,
        scratch_types=[
            pltpu.VMEM((2 * COMBINE_CH,), jnp.int32),
            pltpu.VMEM((2 * COMBINE_CH, D), jnp.float32),
            pltpu.VMEM((COMBINE_CH, D), jnp.float32),
        ] + [pltpu.SemaphoreType.DMA] * (NSTR + 1),
    )(eo, dest)


def kernel(x, Wr, w1, w2, w3):
    bsz, seq, dim = x.shape
    x2 = x.reshape(N, D)
    wr_pad = jnp.zeros((D, 128), jnp.float32).at[:, :E].set(Wr.T)

    top_idx, top_w = _run_router(x2, wr_pad)

    fe = top_idx.reshape(-1)
    oh = (fe[:, None] == jnp.arange(E, dtype=jnp.int32)[None, :]).astype(
        jnp.int32)
    ranks = jnp.cumsum(oh, axis=0) - oh
    rank = jnp.sum(ranks * oh, axis=1)
    counts = jnp.sum(oh, axis=0)
    ps = ((counts + BLK - 1) // BLK) * BLK
    ends = jnp.cumsum(ps)
    po = ends - ps
    dest = (po[fe] + rank).astype(jnp.int32)
    tok = (jnp.arange(A, dtype=jnp.int32) // K)
    gsrc = jnp.zeros((NP,), jnp.int32).at[dest].set(tok)
    ws = jnp.zeros((NP,), jnp.float32).at[dest].set(top_w.reshape(-1))
    block_starts = jnp.arange(NB, dtype=jnp.int32) * BLK
    be = jnp.minimum(
        jnp.sum((block_starts[:, None] >= ends[None, :]).astype(jnp.int32),
                axis=1), E - 1).astype(jnp.int32)

    xg2 = _run_dispatch(x2, gsrc)

    w1b = w1.astype(jnp.bfloat16)
    w3b = w3.astype(jnp.bfloat16)
    w2b = w2.astype(jnp.bfloat16)
    eo = _run_gemm(be, xg2, w1b, w3b, w2b, ws.reshape(NP, 1))

    y = _run_combine(eo, dest)
    return y.reshape(bsz, seq, dim)


# speedup vs baseline: 1.1025x; 1.1025x over previous
import functools

import jax
import jax.numpy as jnp
from jax import lax
from jax.experimental import pallas as pl
from jax.experimental.pallas import tpu as pltpu
from jax.experimental.pallas import tpu_sc as plsc

E = 8
K = 2
D = 2048
F = 512
N = 4096
A = N * K
BLK = 128
NP = A + E * BLK
NB = NP // BLK
RB = 512

NW = 32
DISPATCH_ROWS = NP // NW
DISPATCH_CH = 32
TOK_PER_W = N // NW
COMBINE_CH = 16


def _router_body(x_ref, wr_ref, idx_ref, w_ref):
    x = x_ref[...]
    s = jnp.dot(x, wr_ref[...], preferred_element_type=jnp.float32)
    lane = lax.broadcasted_iota(jnp.int32, (RB, 128), 1)
    neg = jnp.float32(-jnp.inf)
    s = jnp.where(lane < E, s, neg)
    m0 = jnp.max(s, axis=1, keepdims=True)
    i0 = jnp.min(jnp.where(s == m0, lane, 128), axis=1, keepdims=True)
    s2 = jnp.where(lane == i0, neg, s)
    m1 = jnp.max(s2, axis=1, keepdims=True)
    i1 = jnp.min(jnp.where(s2 == m1, lane, 128), axis=1, keepdims=True)
    e1 = jnp.exp(m1 - m0)
    denom = 1.0 + e1
    w0 = 1.0 / denom
    w1 = e1 / denom
    idx_ref[...] = jnp.concatenate([i0, i1], axis=1)
    w_ref[...] = jnp.concatenate([w0, w1], axis=1)


def _run_router(x2, wr_pad):
    return pl.pallas_call(
        _router_body,
        grid=(N // RB,),
        in_specs=[
            pl.BlockSpec((RB, D), lambda i: (i, 0)),
            pl.BlockSpec((D, 128), lambda i: (0, 0)),
        ],
        out_specs=[
            pl.BlockSpec((RB, K), lambda i: (i, 0)),
            pl.BlockSpec((RB, K), lambda i: (i, 0)),
        ],
        out_shape=[
            jax.ShapeDtypeStruct((N, K), jnp.int32),
            jax.ShapeDtypeStruct((N, K), jnp.float32),
        ],
    )(x2, wr_pad)


def _dispatch_body(x_hbm, gsrc_hbm, out_hbm, idx_v, rows_v, sem):
    wid = lax.axis_index("s") * 2 + lax.axis_index("c")
    base = wid * DISPATCH_ROWS
    for c in range(DISPATCH_ROWS // DISPATCH_CH):
        off = base + c * DISPATCH_CH
        pltpu.sync_copy(gsrc_hbm.at[pl.ds(off, DISPATCH_CH)], idx_v)
        pltpu.async_copy(x_hbm.at[idx_v], rows_v, sem).wait()
        pltpu.sync_copy(rows_v, out_hbm.at[pl.ds(off, DISPATCH_CH)])


def _run_dispatch(x2, gsrc):
    mesh = plsc.VectorSubcoreMesh(core_axis_name="c", subcore_axis_name="s")
    return pl.kernel(
        _dispatch_body,
        out_type=jax.ShapeDtypeStruct((NP, D), jnp.float32),
        mesh=mesh,
        scratch_types=[
            pltpu.VMEM((DISPATCH_CH,), jnp.int32),
            pltpu.VMEM((DISPATCH_CH, D), jnp.float32),
            pltpu.SemaphoreType.DMA,
        ],
    )(x2, gsrc)


def _gemm_body(be_ref, xg_ref, w1_ref, w3_ref, w2_ref, ws_ref, out_ref):
    del be_ref
    xb = xg_ref[...].astype(jnp.bfloat16)
    x1 = jnp.dot(xb, w1_ref[0], preferred_element_type=jnp.float32)
    x3 = jnp.dot(xb, w3_ref[0], preferred_element_type=jnp.float32)
    x1 = x1.astype(jnp.bfloat16).astype(jnp.float32)
    x3 = x3.astype(jnp.bfloat16).astype(jnp.float32)
    sg = jax.nn.sigmoid(x1).astype(jnp.bfloat16).astype(jnp.float32)
    sil = (x1 * sg).astype(jnp.bfloat16).astype(jnp.float32)
    h = (sil * x3).astype(jnp.bfloat16)
    out = jnp.dot(h, w2_ref[0], preferred_element_type=jnp.float32)
    out = out.astype(jnp.bfloat16).astype(jnp.float32)
    out_ref[...] = out * ws_ref[...]


def _run_gemm(be, xg2, w1b, w3b, w2b, ws2):
    grid_spec = pltpu.PrefetchScalarGridSpec(
        num_scalar_prefetch=1,
        grid=(NB,),
        in_specs=[
            pl.BlockSpec((BLK, D), lambda i, be: (i, 0)),
            pl.BlockSpec((1, D, F), lambda i, be: (be[i], 0, 0)),
            pl.BlockSpec((1, D, F), lambda i, be: (be[i], 0, 0)),
            pl.BlockSpec((1, F, D), lambda i, be: (be[i], 0, 0)),
            pl.BlockSpec((BLK, 1), lambda i, be: (i, 0)),
        ],
        out_specs=pl.BlockSpec((BLK, D), lambda i, be: (i, 0)),
    )
    return pl.pallas_call(
        _gemm_body,
        grid_spec=grid_spec,
        out_shape=jax.ShapeDtypeStruct((NP, D), jnp.float32),
        compiler_params=pltpu.CompilerParams(
            dimension_semantics=("arbitrary",),
        ),
    )(be, xg2, w1b, w3b, w2b, ws2)


def _combine_body(eo_hbm, dest_hbm, y_hbm, idx_v, rows_v, out_v,
                  gsem, ssem):
    wid = lax.axis_index("s") * 2 + lax.axis_index("c")
    base = wid * TOK_PER_W
    nch = TOK_PER_W // COMBINE_CH
    nrows = 2 * COMBINE_CH
    unroll = 8
    iters_per_row = (D // 16) // unroll
    scp = None

    for c in range(nch):
        off = base + c * COMBINE_CH
        pltpu.sync_copy(dest_hbm.at[pl.ds(2 * off, nrows)], idx_v)
        pltpu.async_copy(eo_hbm.at[idx_v], rows_v, gsem).wait()
        if scp is not None:
            scp.wait()

        def body(i, carry):
            r = i // iters_per_row
            cb = (i % iters_per_row) * unroll
            for j in range(unroll):
                col = (cb + j) * 16
                out_v[r, pl.ds(col, 16)] = (
                    rows_v[2 * r, pl.ds(col, 16)]
                    + rows_v[2 * r + 1, pl.ds(col, 16)]
                )
            return carry

        lax.fori_loop(0, COMBINE_CH * iters_per_row, body, 0)
        scp = pltpu.async_copy(out_v, y_hbm.at[pl.ds(off, COMBINE_CH)], ssem)
    scp.wait()


def _run_combine(eo, dest):
    mesh = plsc.VectorSubcoreMesh(core_axis_name="c", subcore_axis_name="s")
    return pl.kernel(
        _combine_body,
        out_type=jax.ShapeDtypeStruct((N, D), jnp.float32),
        mesh=mesh,
        scratch_types=[
            pltpu.VMEM((2 * COMBINE_CH,), jnp.int32),
            pltpu.VMEM((2 * COMBINE_CH, D), jnp.float32),
            pltpu.VMEM((COMBINE_CH, D), jnp.float32),
            pltpu.SemaphoreType.DMA,
            pltpu.SemaphoreType.DMA,
        ],
    )(eo, dest)


def kernel(x, Wr, w1, w2, w3):
    bsz, seq, dim = x.shape
    x2 = x.reshape(N, D)
    wr_pad = jnp.zeros((D, 128), jnp.float32).at[:, :E].set(Wr.T)

    top_idx, top_w = _run_router(x2, wr_pad)

    fe = top_idx.reshape(-1)
    oh = (fe[:, None] == jnp.arange(E, dtype=jnp.int32)[None, :]).astype(
        jnp.int32)
    ranks = jnp.cumsum(oh, axis=0) - oh
    rank = jnp.sum(ranks * oh, axis=1)
    counts = jnp.sum(oh, axis=0)
    ps = ((counts + BLK - 1) // BLK) * BLK
    ends = jnp.cumsum(ps)
    po = ends - ps
    dest = (po[fe] + rank).astype(jnp.int32)
    tok = (jnp.arange(A, dtype=jnp.int32) // K)
    gsrc = jnp.zeros((NP,), jnp.int32).at[dest].set(tok)
    ws = jnp.zeros((NP,), jnp.float32).at[dest].set(top_w.reshape(-1))
    block_starts = jnp.arange(NB, dtype=jnp.int32) * BLK
    be = jnp.minimum(
        jnp.sum((block_starts[:, None] >= ends[None, :]).astype(jnp.int32),
                axis=1), E - 1).astype(jnp.int32)

    xg2 = _run_dispatch(x2, gsrc)

    w1b = w1.astype(jnp.bfloat16)
    w3b = w3.astype(jnp.bfloat16)
    w2b = w2.astype(jnp.bfloat16)
    eo = _run_gemm(be, xg2, w1b, w3b, w2b, ws.reshape(NP, 1))

    y = _run_combine(eo, dest)
    return y.reshape(bsz, seq, dim)


# speedup vs baseline: 1.2574x; 1.1405x over previous
import functools

import jax
import jax.numpy as jnp
from jax import lax
from jax.experimental import pallas as pl
from jax.experimental.pallas import tpu as pltpu
from jax.experimental.pallas import tpu_sc as plsc

E = 8
K = 2
D = 2048
F = 512
N = 4096
A = N * K
BLK = 256
NP = A + E * BLK
NB = NP // BLK
RB = 512

NW = 32
DISPATCH_ROWS = NP // NW
DISPATCH_CH = 32
TOK_PER_W = N // NW
COMBINE_CH = 16


def _router_body(x_ref, wr_ref, idx_ref, w_ref):
    x = x_ref[...]
    s = jnp.dot(x, wr_ref[...], preferred_element_type=jnp.float32)
    lane = lax.broadcasted_iota(jnp.int32, (RB, 128), 1)
    neg = jnp.float32(-jnp.inf)
    s = jnp.where(lane < E, s, neg)
    m0 = jnp.max(s, axis=1, keepdims=True)
    i0 = jnp.min(jnp.where(s == m0, lane, 128), axis=1, keepdims=True)
    s2 = jnp.where(lane == i0, neg, s)
    m1 = jnp.max(s2, axis=1, keepdims=True)
    i1 = jnp.min(jnp.where(s2 == m1, lane, 128), axis=1, keepdims=True)
    e1 = jnp.exp(m1 - m0)
    denom = 1.0 + e1
    w0 = 1.0 / denom
    w1 = e1 / denom
    idx_ref[...] = jnp.concatenate([i0, i1], axis=1)
    w_ref[...] = jnp.concatenate([w0, w1], axis=1)


def _run_router(x2, wr_pad):
    return pl.pallas_call(
        _router_body,
        grid=(N // RB,),
        in_specs=[
            pl.BlockSpec((RB, D), lambda i: (i, 0)),
            pl.BlockSpec((D, 128), lambda i: (0, 0)),
        ],
        out_specs=[
            pl.BlockSpec((RB, K), lambda i: (i, 0)),
            pl.BlockSpec((RB, K), lambda i: (i, 0)),
        ],
        out_shape=[
            jax.ShapeDtypeStruct((N, K), jnp.int32),
            jax.ShapeDtypeStruct((N, K), jnp.float32),
        ],
    )(x2, wr_pad)


def _dispatch_body(x_hbm, gsrc_hbm, out_hbm, idx_v, rows_v, sem):
    wid = lax.axis_index("s") * 2 + lax.axis_index("c")
    base = wid * DISPATCH_ROWS
    for c in range(DISPATCH_ROWS // DISPATCH_CH):
        off = base + c * DISPATCH_CH
        pltpu.sync_copy(gsrc_hbm.at[pl.ds(off, DISPATCH_CH)], idx_v)
        pltpu.async_copy(x_hbm.at[idx_v], rows_v, sem).wait()
        pltpu.sync_copy(rows_v, out_hbm.at[pl.ds(off, DISPATCH_CH)])


def _run_dispatch(x2, gsrc):
    mesh = plsc.VectorSubcoreMesh(core_axis_name="c", subcore_axis_name="s")
    return pl.kernel(
        _dispatch_body,
        out_type=jax.ShapeDtypeStruct((NP, D), jnp.float32),
        mesh=mesh,
        scratch_types=[
            pltpu.VMEM((DISPATCH_CH,), jnp.int32),
            pltpu.VMEM((DISPATCH_CH, D), jnp.float32),
            pltpu.SemaphoreType.DMA,
        ],
    )(x2, gsrc)


def _gemm_body(be_ref, xg_ref, w1_ref, w3_ref, w2_ref, ws_ref, out_ref):
    del be_ref
    xb = xg_ref[...].astype(jnp.bfloat16)
    x1 = jnp.dot(xb, w1_ref[0], preferred_element_type=jnp.float32)
    x3 = jnp.dot(xb, w3_ref[0], preferred_element_type=jnp.float32)
    x1 = x1.astype(jnp.bfloat16).astype(jnp.float32)
    x3 = x3.astype(jnp.bfloat16).astype(jnp.float32)
    sg = jax.nn.sigmoid(x1).astype(jnp.bfloat16).astype(jnp.float32)
    sil = (x1 * sg).astype(jnp.bfloat16).astype(jnp.float32)
    h = (sil * x3).astype(jnp.bfloat16)
    out = jnp.dot(h, w2_ref[0], preferred_element_type=jnp.float32)
    out = out.astype(jnp.bfloat16).astype(jnp.float32)
    out_ref[...] = out * ws_ref[...]


def _run_gemm(be, xg2, w1b, w3b, w2b, ws2):
    grid_spec = pltpu.PrefetchScalarGridSpec(
        num_scalar_prefetch=1,
        grid=(NB,),
        in_specs=[
            pl.BlockSpec((BLK, D), lambda i, be: (i, 0)),
            pl.BlockSpec((1, D, F), lambda i, be: (be[i], 0, 0)),
            pl.BlockSpec((1, D, F), lambda i, be: (be[i], 0, 0)),
            pl.BlockSpec((1, F, D), lambda i, be: (be[i], 0, 0)),
            pl.BlockSpec((BLK, 1), lambda i, be: (i, 0)),
        ],
        out_specs=pl.BlockSpec((BLK, D), lambda i, be: (i, 0)),
    )
    return pl.pallas_call(
        _gemm_body,
        grid_spec=grid_spec,
        out_shape=jax.ShapeDtypeStruct((NP, D), jnp.float32),
        compiler_params=pltpu.CompilerParams(
            dimension_semantics=("arbitrary",),
        ),
    )(be, xg2, w1b, w3b, w2b, ws2)


def _combine_body(eo_hbm, dest_hbm, y_hbm, idx_v, rows_v, out_v,
                  gsem, ssem):
    wid = lax.axis_index("s") * 2 + lax.axis_index("c")
    base = wid * TOK_PER_W
    nch = TOK_PER_W // COMBINE_CH
    nrows = 2 * COMBINE_CH
    unroll = 8
    iters_per_row = (D // 16) // unroll
    scp = None

    for c in range(nch):
        off = base + c * COMBINE_CH
        pltpu.sync_copy(dest_hbm.at[pl.ds(2 * off, nrows)], idx_v)
        pltpu.async_copy(eo_hbm.at[idx_v], rows_v, gsem).wait()
        if scp is not None:
            scp.wait()

        def body(i, carry):
            r = i // iters_per_row
            cb = (i % iters_per_row) * unroll
            for j in range(unroll):
                col = (cb + j) * 16
                out_v[r, pl.ds(col, 16)] = (
                    rows_v[2 * r, pl.ds(col, 16)]
                    + rows_v[2 * r + 1, pl.ds(col, 16)]
                )
            return carry

        lax.fori_loop(0, COMBINE_CH * iters_per_row, body, 0)
        scp = pltpu.async_copy(out_v, y_hbm.at[pl.ds(off, COMBINE_CH)], ssem)
    scp.wait()


def _run_combine(eo, dest):
    mesh = plsc.VectorSubcoreMesh(core_axis_name="c", subcore_axis_name="s")
    return pl.kernel(
        _combine_body,
        out_type=jax.ShapeDtypeStruct((N, D), jnp.float32),
        mesh=mesh,
        scratch_types=[
            pltpu.VMEM((2 * COMBINE_CH,), jnp.int32),
            pltpu.VMEM((2 * COMBINE_CH, D), jnp.float32),
            pltpu.VMEM((COMBINE_CH, D), jnp.float32),
            pltpu.SemaphoreType.DMA,
            pltpu.SemaphoreType.DMA,
        ],
    )(eo, dest)


def kernel(x, Wr, w1, w2, w3):
    bsz, seq, dim = x.shape
    x2 = x.reshape(N, D)
    wr_pad = jnp.zeros((D, 128), jnp.float32).at[:, :E].set(Wr.T)

    top_idx, top_w = _run_router(x2, wr_pad)

    fe = top_idx.reshape(-1)
    oh = (fe[:, None] == jnp.arange(E, dtype=jnp.int32)[None, :]).astype(
        jnp.int32)
    ranks = jnp.cumsum(oh, axis=0) - oh
    rank = jnp.sum(ranks * oh, axis=1)
    counts = jnp.sum(oh, axis=0)
    ps = ((counts + BLK - 1) // BLK) * BLK
    ends = jnp.cumsum(ps)
    po = ends - ps
    dest = (po[fe] + rank).astype(jnp.int32)
    tok = (jnp.arange(A, dtype=jnp.int32) // K)
    spread = jnp.arange(NP, dtype=jnp.int32) & (N - 1)
    gsrc = spread.at[dest].set(tok)
    ws = jnp.zeros((NP,), jnp.float32).at[dest].set(top_w.reshape(-1))
    block_starts = jnp.arange(NB, dtype=jnp.int32) * BLK
    be = jnp.minimum(
        jnp.sum((block_starts[:, None] >= ends[None, :]).astype(jnp.int32),
                axis=1), E - 1).astype(jnp.int32)

    xg2 = _run_dispatch(x2, gsrc)

    w1b = w1.astype(jnp.bfloat16)
    w3b = w3.astype(jnp.bfloat16)
    w2b = w2.astype(jnp.bfloat16)
    eo = _run_gemm(be, xg2, w1b, w3b, w2b, ws.reshape(NP, 1))

    y = _run_combine(eo, dest)
    return y.reshape(bsz, seq, dim)


# speedup vs baseline: 1.4746x; 1.1727x over previous
import functools

import jax
import jax.numpy as jnp
from jax import lax
from jax.experimental import pallas as pl
from jax.experimental.pallas import tpu as pltpu
from jax.experimental.pallas import tpu_sc as plsc

E = 8
K = 2
D = 2048
F = 512
N = 4096
A = N * K
BLK = 256
NP = A + E * BLK
NB = NP // BLK
RB = 512

NW = 32
DISPATCH_ROWS = NP // NW
DISPATCH_CH = 32
TOK_PER_W = N // NW
COMBINE_CH = 16


def _router_body(x_ref, wr_ref, idx_ref, w_ref):
    x = x_ref[...]
    s = jnp.dot(x, wr_ref[...], preferred_element_type=jnp.float32)
    lane = lax.broadcasted_iota(jnp.int32, (RB, 128), 1)
    neg = jnp.float32(-jnp.inf)
    s = jnp.where(lane < E, s, neg)
    m0 = jnp.max(s, axis=1, keepdims=True)
    i0 = jnp.min(jnp.where(s == m0, lane, 128), axis=1, keepdims=True)
    s2 = jnp.where(lane == i0, neg, s)
    m1 = jnp.max(s2, axis=1, keepdims=True)
    i1 = jnp.min(jnp.where(s2 == m1, lane, 128), axis=1, keepdims=True)
    e1 = jnp.exp(m1 - m0)
    denom = 1.0 + e1
    w0 = 1.0 / denom
    w1 = e1 / denom
    idx_ref[...] = jnp.concatenate([i0, i1], axis=1)
    w_ref[...] = jnp.concatenate([w0, w1], axis=1)


def _run_router(x2, wr_pad):
    return pl.pallas_call(
        _router_body,
        grid=(N // RB,),
        in_specs=[
            pl.BlockSpec((RB, D), lambda i: (i, 0)),
            pl.BlockSpec((D, 128), lambda i: (0, 0)),
        ],
        out_specs=[
            pl.BlockSpec((RB, K), lambda i: (i, 0)),
            pl.BlockSpec((RB, K), lambda i: (i, 0)),
        ],
        out_shape=[
            jax.ShapeDtypeStruct((N, K), jnp.int32),
            jax.ShapeDtypeStruct((N, K), jnp.float32),
        ],
    )(x2, wr_pad)


def _dispatch_body(x_hbm, gsrc_hbm, out_hbm, ia, ib, ra, rb,
                   ga, gb, sa, sb):
    wid = lax.axis_index("s") * 2 + lax.axis_index("c")
    base = wid * DISPATCH_ROWS
    half = DISPATCH_CH // 2
    scpa = scpb = None
    for c in range(DISPATCH_ROWS // DISPATCH_CH):
        off = base + c * DISPATCH_CH
        pltpu.sync_copy(gsrc_hbm.at[pl.ds(off, half)], ia)
        pltpu.sync_copy(gsrc_hbm.at[pl.ds(off + half, half)], ib)
        if scpa is not None:
            scpa.wait()
            scpb.wait()
        cpa = pltpu.async_copy(x_hbm.at[ia], ra, ga)
        cpb = pltpu.async_copy(x_hbm.at[ib], rb, gb)
        cpa.wait()
        cpb.wait()
        scpa = pltpu.async_copy(ra, out_hbm.at[pl.ds(off, half)], sa)
        scpb = pltpu.async_copy(rb, out_hbm.at[pl.ds(off + half, half)], sb)
    scpa.wait()
    scpb.wait()


def _run_dispatch(x2, gsrc):
    mesh = plsc.VectorSubcoreMesh(core_axis_name="c", subcore_axis_name="s")
    return pl.kernel(
        _dispatch_body,
        out_type=jax.ShapeDtypeStruct((NP, D), jnp.float32),
        mesh=mesh,
        scratch_types=[
            pltpu.VMEM((DISPATCH_CH // 2,), jnp.int32),
            pltpu.VMEM((DISPATCH_CH // 2,), jnp.int32),
            pltpu.VMEM((DISPATCH_CH // 2, D), jnp.float32),
            pltpu.VMEM((DISPATCH_CH // 2, D), jnp.float32),
            pltpu.SemaphoreType.DMA,
            pltpu.SemaphoreType.DMA,
            pltpu.SemaphoreType.DMA,
            pltpu.SemaphoreType.DMA,
        ],
    )(x2, gsrc)


def _gemm_body(be_ref, xg_ref, w1_ref, w3_ref, w2_ref, ws_ref, out_ref):
    del be_ref
    xb = xg_ref[...].astype(jnp.bfloat16)
    x1 = jnp.dot(xb, w1_ref[0], preferred_element_type=jnp.float32)
    x3 = jnp.dot(xb, w3_ref[0], preferred_element_type=jnp.float32)
    x1 = x1.astype(jnp.bfloat16).astype(jnp.float32)
    x3 = x3.astype(jnp.bfloat16).astype(jnp.float32)
    sg = jax.nn.sigmoid(x1).astype(jnp.bfloat16).astype(jnp.float32)
    sil = (x1 * sg).astype(jnp.bfloat16).astype(jnp.float32)
    h = (sil * x3).astype(jnp.bfloat16)
    out = jnp.dot(h, w2_ref[0], preferred_element_type=jnp.float32)
    out = out.astype(jnp.bfloat16).astype(jnp.float32)
    out_ref[...] = out * ws_ref[...]


def _run_gemm(be, xg2, w1b, w3b, w2b, ws2):
    grid_spec = pltpu.PrefetchScalarGridSpec(
        num_scalar_prefetch=1,
        grid=(NB,),
        in_specs=[
            pl.BlockSpec((BLK, D), lambda i, be: (i, 0)),
            pl.BlockSpec((1, D, F), lambda i, be: (be[i], 0, 0)),
            pl.BlockSpec((1, D, F), lambda i, be: (be[i], 0, 0)),
            pl.BlockSpec((1, F, D), lambda i, be: (be[i], 0, 0)),
            pl.BlockSpec((BLK, 1), lambda i, be: (i, 0)),
        ],
        out_specs=pl.BlockSpec((BLK, D), lambda i, be: (i, 0)),
    )
    return pl.pallas_call(
        _gemm_body,
        grid_spec=grid_spec,
        out_shape=jax.ShapeDtypeStruct((NP, D), jnp.float32),
        compiler_params=pltpu.CompilerParams(
            dimension_semantics=("arbitrary",),
        ),
    )(be, xg2, w1b, w3b, w2b, ws2)


def _combine_body(eo_hbm, d0_hbm, d1_hbm, y_hbm, i0_v, i1_v, a_v, b_v, o_v,
                  ga, gb, ssem):
    wid = lax.axis_index("s") * 2 + lax.axis_index("c")
    base = wid * TOK_PER_W
    nch = TOK_PER_W // COMBINE_CH
    unroll = 8
    iters_per_row = (D // 16) // unroll
    scp = None

    for c in range(nch):
        off = base + c * COMBINE_CH
        pltpu.sync_copy(d0_hbm.at[pl.ds(off, COMBINE_CH)], i0_v)
        pltpu.sync_copy(d1_hbm.at[pl.ds(off, COMBINE_CH)], i1_v)
        cpa = pltpu.async_copy(eo_hbm.at[i0_v], a_v, ga)
        cpb = pltpu.async_copy(eo_hbm.at[i1_v], b_v, gb)
        cpa.wait()
        cpb.wait()
        if scp is not None:
            scp.wait()

        def body(i, carry):
            r = i // iters_per_row
            cb = (i % iters_per_row) * unroll
            for j in range(unroll):
                col = (cb + j) * 16
                o_v[r, pl.ds(col, 16)] = (
                    a_v[r, pl.ds(col, 16)] + b_v[r, pl.ds(col, 16)]
                )
            return carry

        lax.fori_loop(0, COMBINE_CH * iters_per_row, body, 0)
        scp = pltpu.async_copy(o_v, y_hbm.at[pl.ds(off, COMBINE_CH)], ssem)
    scp.wait()


def _run_combine(eo, d0, d1):
    mesh = plsc.VectorSubcoreMesh(core_axis_name="c", subcore_axis_name="s")
    return pl.kernel(
        _combine_body,
        out_type=jax.ShapeDtypeStruct((N, D), jnp.float32),
        mesh=mesh,
        scratch_types=[
            pltpu.VMEM((COMBINE_CH,), jnp.int32),
            pltpu.VMEM((COMBINE_CH,), jnp.int32),
            pltpu.VMEM((COMBINE_CH, D), jnp.float32),
            pltpu.VMEM((COMBINE_CH, D), jnp.float32),
            pltpu.VMEM((COMBINE_CH, D), jnp.float32),
            pltpu.SemaphoreType.DMA,
            pltpu.SemaphoreType.DMA,
            pltpu.SemaphoreType.DMA,
        ],
    )(eo, d0, d1)


def kernel(x, Wr, w1, w2, w3):
    bsz, seq, dim = x.shape
    x2 = x.reshape(N, D)
    wr_pad = jnp.zeros((D, 128), jnp.float32).at[:, :E].set(Wr.T)

    top_idx, top_w = _run_router(x2, wr_pad)

    fe = top_idx.reshape(-1)
    oh = (fe[:, None] == jnp.arange(E, dtype=jnp.int32)[None, :]).astype(
        jnp.int32)
    ranks = jnp.cumsum(oh, axis=0) - oh
    rank = jnp.sum(ranks * oh, axis=1)
    counts = jnp.sum(oh, axis=0)
    ps = ((counts + BLK - 1) // BLK) * BLK
    ends = jnp.cumsum(ps)
    po = ends - ps
    dest = (po[fe] + rank).astype(jnp.int32)
    tok = (jnp.arange(A, dtype=jnp.int32) // K)
    spread = jnp.arange(NP, dtype=jnp.int32) & (N - 1)
    gsrc = spread.at[dest].set(tok)
    ws = jnp.zeros((NP,), jnp.float32).at[dest].set(top_w.reshape(-1))
    d2 = dest.reshape(N, K)
    d0 = d2[:, 0]
    d1 = d2[:, 1]
    block_starts = jnp.arange(NB, dtype=jnp.int32) * BLK
    be = jnp.minimum(
        jnp.sum((block_starts[:, None] >= ends[None, :]).astype(jnp.int32),
                axis=1), E - 1).astype(jnp.int32)

    xg2 = _run_dispatch(x2, gsrc)

    w1b = w1.astype(jnp.bfloat16)
    w3b = w3.astype(jnp.bfloat16)
    w2b = w2.astype(jnp.bfloat16)
    eo = _run_gemm(be, xg2, w1b, w3b, w2b, ws.reshape(NP, 1))

    y = _run_combine(eo, d0, d1)
    return y.reshape(bsz, seq, dim)


# speedup vs baseline: 1.6181x; 1.0973x over previous
import functools

import jax
import jax.numpy as jnp
from jax import lax
from jax.experimental import pallas as pl
from jax.experimental.pallas import tpu as pltpu
from jax.experimental.pallas import tpu_sc as plsc

E = 8
K = 2
D = 2048
F = 512
N = 4096
A = N * K
BLK = 256
NP = A + E * BLK
NB = NP // BLK
RB = 512

NW = 32
DISPATCH_ROWS = NP // NW
DISPATCH_CH = 32
TOK_PER_W = N // NW
COMBINE_CH = 16


def _router_body(x_ref, wr_ref, idx_ref, w_ref):
    x = x_ref[...]
    s = jnp.dot(x, wr_ref[...], preferred_element_type=jnp.float32)
    lane = lax.broadcasted_iota(jnp.int32, (RB, 128), 1)
    neg = jnp.float32(-jnp.inf)
    s = jnp.where(lane < E, s, neg)
    m0 = jnp.max(s, axis=1, keepdims=True)
    i0 = jnp.min(jnp.where(s == m0, lane, 128), axis=1, keepdims=True)
    s2 = jnp.where(lane == i0, neg, s)
    m1 = jnp.max(s2, axis=1, keepdims=True)
    i1 = jnp.min(jnp.where(s2 == m1, lane, 128), axis=1, keepdims=True)
    e1 = jnp.exp(m1 - m0)
    denom = 1.0 + e1
    w0 = 1.0 / denom
    w1 = e1 / denom
    idx_ref[...] = jnp.concatenate([i0, i1], axis=1)
    w_ref[...] = jnp.concatenate([w0, w1], axis=1)


def _run_router(x2, wr_pad):
    return pl.pallas_call(
        _router_body,
        grid=(N // RB,),
        in_specs=[
            pl.BlockSpec((RB, D), lambda i: (i, 0)),
            pl.BlockSpec((D, 128), lambda i: (0, 0)),
        ],
        out_specs=[
            pl.BlockSpec((RB, K), lambda i: (i, 0)),
            pl.BlockSpec((RB, K), lambda i: (i, 0)),
        ],
        out_shape=[
            jax.ShapeDtypeStruct((N, K), jnp.int32),
            jax.ShapeDtypeStruct((N, K), jnp.float32),
        ],
    )(x2, wr_pad)


def _dispatch_body(x_hbm, gsrc_hbm, out_hbm, ia, ib, ra, rb,
                   ga, gb, sa, sb):
    wid = lax.axis_index("s") * 2 + lax.axis_index("c")
    base = wid * DISPATCH_ROWS
    half = DISPATCH_CH // 2
    scpa = scpb = None
    for c in range(DISPATCH_ROWS // DISPATCH_CH):
        off = base + c * DISPATCH_CH
        pltpu.sync_copy(gsrc_hbm.at[pl.ds(off, half)], ia)
        pltpu.sync_copy(gsrc_hbm.at[pl.ds(off + half, half)], ib)
        if scpa is not None:
            scpa.wait()
            scpb.wait()
        cpa = pltpu.async_copy(x_hbm.at[ia], ra, ga)
        cpb = pltpu.async_copy(x_hbm.at[ib], rb, gb)
        cpa.wait()
        cpb.wait()
        scpa = pltpu.async_copy(ra, out_hbm.at[pl.ds(off, half)], sa)
        scpb = pltpu.async_copy(rb, out_hbm.at[pl.ds(off + half, half)], sb)
    scpa.wait()
    scpb.wait()


def _run_dispatch(x2, gsrc):
    mesh = plsc.VectorSubcoreMesh(core_axis_name="c", subcore_axis_name="s")
    return pl.kernel(
        _dispatch_body,
        out_type=jax.ShapeDtypeStruct((NP, D), jnp.float32),
        mesh=mesh,
        scratch_types=[
            pltpu.VMEM((DISPATCH_CH // 2,), jnp.int32),
            pltpu.VMEM((DISPATCH_CH // 2,), jnp.int32),
            pltpu.VMEM((DISPATCH_CH // 2, D), jnp.float32),
            pltpu.VMEM((DISPATCH_CH // 2, D), jnp.float32),
            pltpu.SemaphoreType.DMA,
            pltpu.SemaphoreType.DMA,
            pltpu.SemaphoreType.DMA,
            pltpu.SemaphoreType.DMA,
        ],
    )(x2, gsrc)


def _gemm_body(be_ref, xg_ref, w1_ref, w3_ref, w2_ref, ws_ref, out_ref):
    del be_ref
    xb = xg_ref[...].astype(jnp.bfloat16)
    x1 = jnp.dot(xb, w1_ref[0].astype(jnp.bfloat16),
                 preferred_element_type=jnp.float32)
    x3 = jnp.dot(xb, w3_ref[0].astype(jnp.bfloat16),
                 preferred_element_type=jnp.float32)
    x1 = x1.astype(jnp.bfloat16).astype(jnp.float32)
    x3 = x3.astype(jnp.bfloat16).astype(jnp.float32)
    sg = jax.nn.sigmoid(x1).astype(jnp.bfloat16).astype(jnp.float32)
    sil = (x1 * sg).astype(jnp.bfloat16).astype(jnp.float32)
    h = (sil * x3).astype(jnp.bfloat16)
    out = jnp.dot(h, w2_ref[0].astype(jnp.bfloat16),
                  preferred_element_type=jnp.float32)
    out = out.astype(jnp.bfloat16).astype(jnp.float32)
    out_ref[...] = out * ws_ref[...]


def _run_gemm(be, xg2, w1b, w3b, w2b, ws2):
    grid_spec = pltpu.PrefetchScalarGridSpec(
        num_scalar_prefetch=1,
        grid=(NB,),
        in_specs=[
            pl.BlockSpec((BLK, D), lambda i, be: (i, 0)),
            pl.BlockSpec((1, D, F), lambda i, be: (be[i], 0, 0)),
            pl.BlockSpec((1, D, F), lambda i, be: (be[i], 0, 0)),
            pl.BlockSpec((1, F, D), lambda i, be: (be[i], 0, 0)),
            pl.BlockSpec((BLK, 1), lambda i, be: (i, 0)),
        ],
        out_specs=pl.BlockSpec((BLK, D), lambda i, be: (i, 0)),
    )
    return pl.pallas_call(
        _gemm_body,
        grid_spec=grid_spec,
        out_shape=jax.ShapeDtypeStruct((NP, D), jnp.float32),
        compiler_params=pltpu.CompilerParams(
            dimension_semantics=("arbitrary",),
        ),
    )(be, xg2, w1b, w3b, w2b, ws2)


def _combine_body(eo_hbm, d0_hbm, d1_hbm, y_hbm, i0_v, i1_v, a_v, b_v, o_v,
                  ga, gb, ssem):
    wid = lax.axis_index("s") * 2 + lax.axis_index("c")
    base = wid * TOK_PER_W
    nch = TOK_PER_W // COMBINE_CH
    unroll = 8
    iters_per_row = (D // 16) // unroll
    scp = None

    for c in range(nch):
        off = base + c * COMBINE_CH
        pltpu.sync_copy(d0_hbm.at[pl.ds(off, COMBINE_CH)], i0_v)
        pltpu.sync_copy(d1_hbm.at[pl.ds(off, COMBINE_CH)], i1_v)
        cpa = pltpu.async_copy(eo_hbm.at[i0_v], a_v, ga)
        cpb = pltpu.async_copy(eo_hbm.at[i1_v], b_v, gb)
        cpa.wait()
        cpb.wait()
        if scp is not None:
            scp.wait()

        def body(i, carry):
            r = i // iters_per_row
            cb = (i % iters_per_row) * unroll
            for j in range(unroll):
                col = (cb + j) * 16
                o_v[r, pl.ds(col, 16)] = (
                    a_v[r, pl.ds(col, 16)] + b_v[r, pl.ds(col, 16)]
                )
            return carry

        lax.fori_loop(0, COMBINE_CH * iters_per_row, body, 0)
        scp = pltpu.async_copy(o_v, y_hbm.at[pl.ds(off, COMBINE_CH)], ssem)
    scp.wait()


def _run_combine(eo, d0, d1):
    mesh = plsc.VectorSubcoreMesh(core_axis_name="c", subcore_axis_name="s")
    return pl.kernel(
        _combine_body,
        out_type=jax.ShapeDtypeStruct((N, D), jnp.float32),
        mesh=mesh,
        scratch_types=[
            pltpu.VMEM((COMBINE_CH,), jnp.int32),
            pltpu.VMEM((COMBINE_CH,), jnp.int32),
            pltpu.VMEM((COMBINE_CH, D), jnp.float32),
            pltpu.VMEM((COMBINE_CH, D), jnp.float32),
            pltpu.VMEM((COMBINE_CH, D), jnp.float32),
            pltpu.SemaphoreType.DMA,
            pltpu.SemaphoreType.DMA,
            pltpu.SemaphoreType.DMA,
        ],
    )(eo, d0, d1)


def kernel(x, Wr, w1, w2, w3):
    bsz, seq, dim = x.shape
    x2 = x.reshape(N, D)
    wr_pad = jnp.zeros((D, 128), jnp.float32).at[:, :E].set(Wr.T)

    top_idx, top_w = _run_router(x2, wr_pad)

    fe = top_idx.reshape(-1)
    oh = (fe[:, None] == jnp.arange(E, dtype=jnp.int32)[None, :]).astype(
        jnp.int32)
    ranks = jnp.cumsum(oh, axis=0) - oh
    rank = jnp.sum(ranks * oh, axis=1)
    counts = jnp.sum(oh, axis=0)
    ps = ((counts + BLK - 1) // BLK) * BLK
    ends = jnp.cumsum(ps)
    po = ends - ps
    dest = (po[fe] + rank).astype(jnp.int32)
    tok = (jnp.arange(A, dtype=jnp.int32) // K)
    spread = jnp.arange(NP, dtype=jnp.int32) & (N - 1)
    gsrc = spread.at[dest].set(tok)
    ws = jnp.zeros((NP,), jnp.float32).at[dest].set(top_w.reshape(-1))
    d2 = dest.reshape(N, K)
    d0 = d2[:, 0]
    d1 = d2[:, 1]
    block_starts = jnp.arange(NB, dtype=jnp.int32) * BLK
    be = jnp.minimum(
        jnp.sum((block_starts[:, None] >= ends[None, :]).astype(jnp.int32),
                axis=1), E - 1).astype(jnp.int32)

    xg2 = _run_dispatch(x2, gsrc)

    eo = _run_gemm(be, xg2, w1, w3, w2, ws.reshape(NP, 1))

    y = _run_combine(eo, d0, d1)
    return y.reshape(bsz, seq, dim)
